# baseline (speedup 1.0000x reference)
import dataclasses
import functools

import jax
import jax.numpy as jnp
from jax import lax
from jax.experimental import pallas as pl
from jax.experimental.pallas import tpu as pltpu
from jax.experimental.pallas import tpu_sc as plsc

M = 1000000
D = 128
B = 16384
DECAY_F = 0.95

NC = 2
NS = 16
L = 16
NW = NC * NS
CHUNK = 128
ROWS = B // CHUNK
H = B // NC
TRASH = H

_mesh = plsc.VectorSubcoreMesh(core_axis_name="c", subcore_axis_name="s")

_no_layout_cp = pltpu.CompilerParams()
if "needs_layout_passes" in pltpu.CompilerParams.__dataclass_fields__:
    _no_layout_cp = dataclasses.replace(_no_layout_cp, needs_layout_passes=False)


def _wid():
    return lax.axis_index("s") * NC + lax.axis_index("c")


@functools.partial(
    pl.kernel,
    out_type=jax.ShapeDtypeStruct((M,), jnp.int32),
    mesh=_mesh,
    scratch_types=[
        pltpu.VMEM((ROWS // NW, CHUNK), jnp.int32),
        pltpu.VMEM((ROWS // NW, CHUNK), jnp.int32),
        pltpu.SemaphoreType.DMA,
        pltpu.SemaphoreType.DMA,
    ],
)
def _tag_kernel(idx_hbm, tag_hbm, idxb, jb, sem_in, sem_sc):
    wid = _wid()
    n_per = ROWS // NW
    row0 = wid * n_per

    cp = pltpu.async_copy(idx_hbm.at[pl.ds(row0, n_per)], idxb, sem_in)
    for q in range(n_per):
        j0 = (row0 + q) * CHUNK
        for l in range(0, CHUNK, L):
            jb.at[q, pl.ds(l, L)][...] = j0 + l + lax.iota(jnp.int32, L)
    cp.wait()
    cps = [
        pltpu.async_copy(jb.at[q], tag_hbm.at[idxb.at[q]], sem_sc)
        for q in range(n_per)
    ]
    for cp in cps:
        cp.wait()


@functools.partial(
    pl.kernel,
    out_type=jax.ShapeDtypeStruct((B + 8, D), jnp.float32),
    mesh=_mesh,
    scratch_types=[
        pltpu.VMEM((CHUNK, D), jnp.float32),
        pltpu.VMEM((CHUNK, D), jnp.float32),
        pltpu.VMEM((CHUNK, D), jnp.float32),
        pltpu.VMEM((ROWS // NS, CHUNK), jnp.int32),
        pltpu.VMEM((ROWS // NS, CHUNK), jnp.int32),
        pltpu.VMEM((ROWS // NS, CHUNK), jnp.int32),
        pltpu.VMEM((ROWS // NS, CHUNK), jnp.int32),
        pltpu.VMEM((ROWS // NS, CHUNK), jnp.int32),
        pltpu.VMEM((ROWS // NS, CHUNK), jnp.int32),
        pltpu.VMEM_SHARED((H + 8, D), jnp.float32),
        pltpu.SemaphoreType.DMA,
        pltpu.SemaphoreType.DMA,
        pltpu.SemaphoreType.DMA,
    ],
    compiler_params=_no_layout_cp,
)
def _acc_kernel(idx_hbm, tag_hbm, val_hbm, acc_hbm,
                vb0, vb1, vb2, idxb, wb, jl, gl, tl, dl, acc_sh,
                sem_in, sem_cp, sem_wr):
    c = lax.axis_index("c")
    s = lax.axis_index("s")
    half0 = c * H
    n_per = ROWS // NS
    row0 = s * n_per
    crows = H // NS

    cp_idx = pltpu.async_copy(idx_hbm.at[pl.ds(row0, n_per)], idxb, sem_in)

    vbufs = (vb0, vb1, vb2)
    n_cp = crows // CHUNK
    cin = [None] * n_cp
    cout = [None] * n_cp
    for q in range(min(3, n_cp)):
        cin[q] = pltpu.async_copy(
            val_hbm.at[pl.ds(half0 + s * crows + q * CHUNK, CHUNK)],
            vbufs[q % 3], sem_cp)
    for q in range(n_cp):
        if q >= 3:
            cout[q - 3].wait()
            cin[q] = pltpu.async_copy(
                val_hbm.at[pl.ds(half0 + s * crows + q * CHUNK, CHUNK)],
                vbufs[q % 3], sem_cp)
        cin[q].wait()
        cout[q] = pltpu.async_copy(
            vbufs[q % 3],
            acc_hbm.at[pl.ds(half0 + s * crows + q * CHUNK, CHUNK)], sem_wr)
    copy_cps = [cout[q] for q in range(max(0, n_cp - 3), n_cp)]
    cp_idx.wait()
    tag_cps = [
        pltpu.async_copy(tag_hbm.at[idxb.at[q]], wb.at[q], sem_in)
        for q in range(n_per)
    ]

    zero_v = jnp.zeros((L,), jnp.int32)
    for q in range(n_per):
        for l in range(0, CHUNK, L):
            jl.at[q, pl.ds(l, L)][...] = zero_v
            gl.at[q, pl.ds(l, L)][...] = zero_v
            dl.at[q, pl.ds(l, L)][...] = zero_v + B
            tl.at[q, pl.ds(l, L)][...] = zero_v + TRASH
    for cp in tag_cps:
        cp.wait()

    iota_v = lax.iota(jnp.int32, L)
    cnt = jnp.int32(0)
    for q in range(n_per):
        for l in range(0, CHUNK, L):
            jv = (row0 + q) * CHUNK + l + iota_v
            wv = wb.at[q, pl.ds(l, L)][...]
            local = wv - half0
            lose = (wv != jv) & (local >= 0) & (local < H)
            mi = lose.astype(jnp.int32)
            pc = plsc.cumsum(mi)
            pos = cnt + pc - 1
            plsc.store_scatter(jl, [pos >> 7, pos & 127], jv, mask=lose)
            plsc.store_scatter(gl, [pos >> 7, pos & 127], wv, mask=lose)
            plsc.store_scatter(dl, [pos >> 7, pos & 127], wv, mask=lose)
            plsc.store_scatter(tl, [pos >> 7, pos & 127], local, mask=lose)
            cnt = cnt + jnp.sum(mi)
    nblk = (cnt + CHUNK - 1) // CHUNK
    nblk = jnp.int32(1)
    for cp in copy_cps:
        cp.wait()
    plsc.subcore_barrier()

    for b in range(n_per):
        @pl.when(b < nblk)
        def _():
            pltpu.sync_copy(val_hbm.at[gl.at[b]], vb0)
            pltpu.sync_copy(vb0, acc_sh.at[tl.at[b]])

    plsc.subcore_barrier()

    for b in range(n_per):
        @pl.when(b < nblk)
        def _():
            pltpu.sync_copy(val_hbm.at[jl.at[b]], vb1)
            pltpu.sync_copy(vb1, acc_sh.at[tl.at[b]], add=True)

    plsc.subcore_barrier()

    for b in range(n_per):
        @pl.when(b < nblk)
        def _():
            pltpu.sync_copy(acc_sh.at[tl.at[b]], vb2)
            pltpu.sync_copy(vb2, acc_hbm.at[dl.at[b]])


@functools.partial(
    pl.kernel,
    out_type=jax.ShapeDtypeStruct((B, D), jnp.float32),
    mesh=_mesh,
    scratch_types=[
        pltpu.VMEM((ROWS // NW, CHUNK), jnp.int32),
        pltpu.VMEM((ROWS // NW, CHUNK), jnp.int32),
        pltpu.VMEM((CHUNK, D), jnp.float32),
        pltpu.VMEM((CHUNK, D), jnp.float32),
        pltpu.VMEM((CHUNK, D), jnp.float32),
        pltpu.VMEM((CHUNK, D), jnp.float32),
        pltpu.SemaphoreType.DMA,
        pltpu.SemaphoreType.DMA,
        pltpu.SemaphoreType.DMA,
    ],
)
def _combine_kernel(idx_hbm, tag_hbm, mem_hbm, acc_hbm, out_hbm,
                    idxb, wb, mb0, mb1, ab0, ab1, sem_in, sem_g, sem_out):
    wid = _wid()
    n_per = ROWS // NW
    row0 = wid * n_per

    pltpu.sync_copy(idx_hbm.at[pl.ds(row0, n_per)], idxb)
    tag_cps = [
        pltpu.async_copy(tag_hbm.at[idxb.at[q]], wb.at[q], sem_in)
        for q in range(n_per)
    ]
    for cp in tag_cps:
        cp.wait()

    mbufs = (mb0, mb1)
    abufs = (ab0, ab1)

    def fire(q):
        b = q % 2
        return (pltpu.async_copy(mem_hbm.at[idxb.at[q]], mbufs[b], sem_g),
                pltpu.async_copy(acc_hbm.at[wb.at[q]], abufs[b], sem_g))

    g_cps = [None] * n_per
    out_cps = [None] * n_per
    g_cps[0] = fire(0)
    for q in range(n_per):
        mb, ab = mbufs[q % 2], abufs[q % 2]
        if q + 1 < n_per:
            if q >= 1:
                out_cps[q - 1].wait()
            g_cps[q + 1] = fire(q + 1)
        for cp in g_cps[q]:
            cp.wait()

        @pl.loop(0, CHUNK)
        def _(r):
            for l in range(0, D, L):
                sl = (r, pl.ds(l, L))
                mb.at[sl][...] = mb.at[sl][...] + ab.at[sl][...] * DECAY_F

        out_cps[q] = pltpu.async_copy(
            mb, out_hbm.at[pl.ds((row0 + q) * CHUNK, CHUNK)], sem_out)
    out_cps[n_per - 2].wait()
    out_cps[n_per - 1].wait()


def kernel(mem, idx, val):
    idx2 = jnp.reshape(idx.astype(jnp.int32), (ROWS, CHUNK))
    tags = _tag_kernel(idx2)
    acc = _acc_kernel(idx2, tags, val)
    return _combine_kernel(idx2, tags, mem, acc)


# speedup vs baseline: 6.2735x; 6.2735x over previous
import dataclasses
import functools

import jax
import jax.numpy as jnp
from jax import lax
from jax.experimental import pallas as pl
from jax.experimental.pallas import tpu as pltpu
from jax.experimental.pallas import tpu_sc as plsc

M = 1000000
D = 128
B = 16384
DECAY_F = 0.95

NC = 2
NS = 16
L = 16
NW = NC * NS
CHUNK = 128
ROWS = B // CHUNK
H = B // NC
TRASH = H

_mesh = plsc.VectorSubcoreMesh(core_axis_name="c", subcore_axis_name="s")

_no_layout_cp = pltpu.CompilerParams()
if "needs_layout_passes" in pltpu.CompilerParams.__dataclass_fields__:
    _no_layout_cp = dataclasses.replace(_no_layout_cp, needs_layout_passes=False)


def _wid():
    return lax.axis_index("s") * NC + lax.axis_index("c")


@functools.partial(
    pl.kernel,
    out_type=jax.ShapeDtypeStruct((M,), jnp.int32),
    mesh=_mesh,
    scratch_types=[
        pltpu.VMEM((ROWS // NW, CHUNK), jnp.int32),
        pltpu.VMEM((ROWS // NW, CHUNK), jnp.int32),
        pltpu.SemaphoreType.DMA,
        pltpu.SemaphoreType.DMA,
    ],
)
def _tag_kernel(idx_hbm, tag_hbm, idxb, jb, sem_in, sem_sc):
    wid = _wid()
    n_per = ROWS // NW
    row0 = wid * n_per

    cp = pltpu.async_copy(idx_hbm.at[pl.ds(row0, n_per)], idxb, sem_in)
    for q in range(n_per):
        j0 = (row0 + q) * CHUNK
        for l in range(0, CHUNK, L):
            jb.at[q, pl.ds(l, L)][...] = j0 + l + lax.iota(jnp.int32, L)
    cp.wait()
    cps = [
        pltpu.async_copy(jb.at[q], tag_hbm.at[idxb.at[q]], sem_sc)
        for q in range(n_per)
    ]
    for cp in cps:
        cp.wait()


@functools.partial(
    pl.kernel,
    out_type=jax.ShapeDtypeStruct((B + CHUNK, D), jnp.float32),
    mesh=_mesh,
    scratch_types=[
        pltpu.VMEM((CHUNK, D), jnp.float32),
        pltpu.VMEM((CHUNK, D), jnp.float32),
        pltpu.VMEM((CHUNK, D), jnp.float32),
        pltpu.VMEM((ROWS // NS, CHUNK), jnp.int32),
        pltpu.VMEM((ROWS // NS, CHUNK), jnp.int32),
        pltpu.VMEM((ROWS // NS, CHUNK), jnp.int32),
        pltpu.VMEM((ROWS // NS, CHUNK), jnp.int32),
        pltpu.VMEM((ROWS // NS, CHUNK), jnp.int32),
        pltpu.VMEM((ROWS // NS, CHUNK), jnp.int32),
        pltpu.VMEM_SHARED((H + CHUNK, D), jnp.float32),
        pltpu.SemaphoreType.DMA,
        pltpu.SemaphoreType.DMA,
        pltpu.SemaphoreType.DMA,
    ],
    compiler_params=_no_layout_cp,
)
def _acc_kernel(idx_hbm, tag_hbm, val_hbm, acc_hbm,
                vb0, vb1, vb2, idxb, wb, jl, gl, tl, dl, acc_sh,
                sem_in, sem_cp, sem_wr):
    c = lax.axis_index("c")
    s = lax.axis_index("s")
    half0 = c * H
    n_per = ROWS // NS
    row0 = s * n_per
    crows = H // NS

    cp_idx = pltpu.async_copy(idx_hbm.at[pl.ds(row0, n_per)], idxb, sem_in)

    vbufs = (vb0, vb1, vb2)
    n_cp = crows // CHUNK
    cin = [None] * n_cp
    cout = [None] * n_cp
    for q in range(min(3, n_cp)):
        cin[q] = pltpu.async_copy(
            val_hbm.at[pl.ds(half0 + s * crows + q * CHUNK, CHUNK)],
            vbufs[q % 3], sem_cp)
    for q in range(n_cp):
        if q >= 3:
            cout[q - 3].wait()
            cin[q] = pltpu.async_copy(
                val_hbm.at[pl.ds(half0 + s * crows + q * CHUNK, CHUNK)],
                vbufs[q % 3], sem_cp)
        cin[q].wait()
        cout[q] = pltpu.async_copy(
            vbufs[q % 3],
            acc_hbm.at[pl.ds(half0 + s * crows + q * CHUNK, CHUNK)], sem_wr)
    copy_cps = [cout[q] for q in range(max(0, n_cp - 3), n_cp)]
    cp_idx.wait()
    tag_cps = [
        pltpu.async_copy(tag_hbm.at[idxb.at[q]], wb.at[q], sem_in)
        for q in range(n_per)
    ]

    iota_v = lax.iota(jnp.int32, L)
    for q in range(n_per):
        for l in range(0, CHUNK, L):
            pad = l + iota_v
            jl.at[q, pl.ds(l, L)][...] = pad
            gl.at[q, pl.ds(l, L)][...] = pad
            dl.at[q, pl.ds(l, L)][...] = pad + B
            tl.at[q, pl.ds(l, L)][...] = pad + TRASH
    for cp in tag_cps:
        cp.wait()

    cnt = jnp.int32(0)
    for q in range(n_per):
        for l in range(0, CHUNK, L):
            jv = (row0 + q) * CHUNK + l + iota_v
            wv = wb.at[q, pl.ds(l, L)][...]
            local = wv - half0
            lose = (wv != jv) & (local >= 0) & (local < H)
            mi = lose.astype(jnp.int32)
            pc = plsc.cumsum(mi)
            pos = cnt + pc - 1
            plsc.store_scatter(jl, [pos >> 7, pos & 127], jv, mask=lose)
            plsc.store_scatter(gl, [pos >> 7, pos & 127], wv, mask=lose)
            plsc.store_scatter(dl, [pos >> 7, pos & 127], wv, mask=lose)
            plsc.store_scatter(tl, [pos >> 7, pos & 127], local, mask=lose)
            cnt = cnt + jnp.sum(mi)
    nblk = (cnt + CHUNK - 1) // CHUNK
    for cp in copy_cps:
        cp.wait()
    plsc.subcore_barrier()

    for b in range(n_per):
        @pl.when(b < nblk)
        def _():
            pltpu.sync_copy(val_hbm.at[gl.at[b]], vb0)
            pltpu.sync_copy(vb0, acc_sh.at[tl.at[b]])

    plsc.subcore_barrier()

    for b in range(n_per):
        @pl.when(b < nblk)
        def _():
            pltpu.sync_copy(val_hbm.at[jl.at[b]], vb1)
            pltpu.sync_copy(vb1, acc_sh.at[tl.at[b]], add=True)

    plsc.subcore_barrier()

    for b in range(n_per):
        @pl.when(b < nblk)
        def _():
            pltpu.sync_copy(acc_sh.at[tl.at[b]], vb2)
            pltpu.sync_copy(vb2, acc_hbm.at[dl.at[b]])


@functools.partial(
    pl.kernel,
    out_type=jax.ShapeDtypeStruct((B, D), jnp.float32),
    mesh=_mesh,
    scratch_types=[
        pltpu.VMEM((ROWS // NW, CHUNK), jnp.int32),
        pltpu.VMEM((ROWS // NW, CHUNK), jnp.int32),
        pltpu.VMEM((CHUNK, D), jnp.float32),
        pltpu.VMEM((CHUNK, D), jnp.float32),
        pltpu.VMEM((CHUNK, D), jnp.float32),
        pltpu.VMEM((CHUNK, D), jnp.float32),
        pltpu.SemaphoreType.DMA,
        pltpu.SemaphoreType.DMA,
        pltpu.SemaphoreType.DMA,
    ],
)
def _combine_kernel(idx_hbm, tag_hbm, mem_hbm, acc_hbm, out_hbm,
                    idxb, wb, mb0, mb1, ab0, ab1, sem_in, sem_g, sem_out):
    wid = _wid()
    n_per = ROWS // NW
    row0 = wid * n_per

    pltpu.sync_copy(idx_hbm.at[pl.ds(row0, n_per)], idxb)
    tag_cps = [
        pltpu.async_copy(tag_hbm.at[idxb.at[q]], wb.at[q], sem_in)
        for q in range(n_per)
    ]
    for cp in tag_cps:
        cp.wait()

    mbufs = (mb0, mb1)
    abufs = (ab0, ab1)

    def fire(q):
        b = q % 2
        return (pltpu.async_copy(mem_hbm.at[idxb.at[q]], mbufs[b], sem_g),
                pltpu.async_copy(acc_hbm.at[wb.at[q]], abufs[b], sem_g))

    g_cps = [None] * n_per
    out_cps = [None] * n_per
    g_cps[0] = fire(0)
    for q in range(n_per):
        mb, ab = mbufs[q % 2], abufs[q % 2]
        if q + 1 < n_per:
            if q >= 1:
                out_cps[q - 1].wait()
            g_cps[q + 1] = fire(q + 1)
        for cp in g_cps[q]:
            cp.wait()

        @pl.loop(0, CHUNK)
        def _(r):
            for l in range(0, D, L):
                sl = (r, pl.ds(l, L))
                mb.at[sl][...] = mb.at[sl][...] + ab.at[sl][...] * DECAY_F

        out_cps[q] = pltpu.async_copy(
            mb, out_hbm.at[pl.ds((row0 + q) * CHUNK, CHUNK)], sem_out)
    out_cps[n_per - 2].wait()
    out_cps[n_per - 1].wait()


def kernel(mem, idx, val):
    idx2 = jnp.reshape(idx.astype(jnp.int32), (ROWS, CHUNK))
    tags = _tag_kernel(idx2)
    acc = _acc_kernel(idx2, tags, val)
    return _combine_kernel(idx2, tags, mem, acc)


# speedup vs baseline: 6.6958x; 1.0673x over previous
import dataclasses
import functools

import jax
import jax.numpy as jnp
from jax import lax
from jax.experimental import pallas as pl
from jax.experimental.pallas import tpu as pltpu
from jax.experimental.pallas import tpu_sc as plsc

M = 1000000
D = 128
B = 16384
DECAY_F = 0.95

NC = 2
NS = 16
L = 16
NW = NC * NS
CHUNK = 128
ROWS = B // CHUNK
H = B // NC
TRASH = H

_mesh = plsc.VectorSubcoreMesh(core_axis_name="c", subcore_axis_name="s")

_no_layout_cp = pltpu.CompilerParams()
if "needs_layout_passes" in pltpu.CompilerParams.__dataclass_fields__:
    _no_layout_cp = dataclasses.replace(_no_layout_cp, needs_layout_passes=False)


def _wid():
    return lax.axis_index("s") * NC + lax.axis_index("c")


@functools.partial(
    pl.kernel,
    out_type=jax.ShapeDtypeStruct((M,), jnp.int32),
    mesh=_mesh,
    scratch_types=[
        pltpu.VMEM((ROWS // NW, CHUNK), jnp.int32),
        pltpu.VMEM((ROWS // NW, CHUNK), jnp.int32),
        pltpu.SemaphoreType.DMA,
        pltpu.SemaphoreType.DMA,
    ],
)
def _tag_kernel(idx_hbm, tag_hbm, idxb, jb, sem_in, sem_sc):
    wid = _wid()
    n_per = ROWS // NW
    row0 = wid * n_per

    cp = pltpu.async_copy(idx_hbm.at[pl.ds(row0, n_per)], idxb, sem_in)
    for q in range(n_per):
        j0 = (row0 + q) * CHUNK
        for l in range(0, CHUNK, L):
            jb.at[q, pl.ds(l, L)][...] = j0 + l + lax.iota(jnp.int32, L)
    cp.wait()
    cps = [
        pltpu.async_copy(jb.at[q], tag_hbm.at[idxb.at[q]], sem_sc)
        for q in range(n_per)
    ]
    for cp in cps:
        cp.wait()


@functools.partial(
    pl.kernel,
    out_type=jax.ShapeDtypeStruct((B + CHUNK, D), jnp.float32),
    mesh=_mesh,
    scratch_types=[
        pltpu.VMEM((CHUNK, D), jnp.float32),
        pltpu.VMEM((CHUNK, D), jnp.float32),
        pltpu.VMEM((CHUNK, D), jnp.float32),
        pltpu.VMEM((ROWS // NS, CHUNK), jnp.int32),
        pltpu.VMEM((ROWS // NS, CHUNK), jnp.int32),
        pltpu.VMEM((ROWS // NS, CHUNK), jnp.int32),
        pltpu.VMEM((ROWS // NS, CHUNK), jnp.int32),
        pltpu.VMEM((ROWS // NS, CHUNK), jnp.int32),
        pltpu.VMEM((ROWS // NS, CHUNK), jnp.int32),
        pltpu.VMEM((1, 32), jnp.int32),
        pltpu.VMEM((1, 32), jnp.int32),
        pltpu.VMEM((1, 32), jnp.int32),
        pltpu.VMEM((1, 32), jnp.int32),
        pltpu.VMEM_SHARED((H + CHUNK, D), jnp.float32),
        pltpu.SemaphoreType.DMA,
        pltpu.SemaphoreType.DMA,
        pltpu.SemaphoreType.DMA,
    ],
    compiler_params=_no_layout_cp,
)
def _acc_kernel(idx_hbm, tag_hbm, val_hbm, acc_hbm,
                vb0, vb1, vb2, idxb, wb, jl, gl, tl, dl,
                jl1, gl1, dl1, tl1, acc_sh,
                sem_in, sem_cp, sem_wr):
    c = lax.axis_index("c")
    s = lax.axis_index("s")
    half0 = c * H
    n_per = ROWS // NS
    row0 = s * n_per
    crows = H // NS

    cp_idx = pltpu.async_copy(idx_hbm.at[pl.ds(row0, n_per)], idxb, sem_in)

    vbufs = (vb0, vb1, vb2)
    n_cp = crows // CHUNK
    cin = [None] * n_cp
    cout = [None] * n_cp
    for q in range(min(3, n_cp)):
        cin[q] = pltpu.async_copy(
            val_hbm.at[pl.ds(half0 + s * crows + q * CHUNK, CHUNK)],
            vbufs[q % 3], sem_cp)
    for q in range(n_cp):
        if q >= 3:
            cout[q - 3].wait()
            cin[q] = pltpu.async_copy(
                val_hbm.at[pl.ds(half0 + s * crows + q * CHUNK, CHUNK)],
                vbufs[q % 3], sem_cp)
        cin[q].wait()
        cout[q] = pltpu.async_copy(
            vbufs[q % 3],
            acc_hbm.at[pl.ds(half0 + s * crows + q * CHUNK, CHUNK)], sem_wr)
    copy_cps = [cout[q] for q in range(max(0, n_cp - 3), n_cp)]
    cp_idx.wait()
    tag_cps = [
        pltpu.async_copy(tag_hbm.at[idxb.at[q]], wb.at[q], sem_in)
        for q in range(n_per)
    ]

    iota_v = lax.iota(jnp.int32, L)
    iota_v = lax.iota(jnp.int32, L)
    for q in range(n_per):
        for l in range(0, CHUNK, L):
            pad = l + iota_v
            jl.at[q, pl.ds(l, L)][...] = pad
            gl.at[q, pl.ds(l, L)][...] = pad
            dl.at[q, pl.ds(l, L)][...] = pad + B
            tl.at[q, pl.ds(l, L)][...] = pad + TRASH
    for l in range(0, 32, L):
        pad = l + iota_v
        jl1.at[0, pl.ds(l, L)][...] = pad
        gl1.at[0, pl.ds(l, L)][...] = pad
        dl1.at[0, pl.ds(l, L)][...] = pad + B
        tl1.at[0, pl.ds(l, L)][...] = pad + TRASH
    for cp in tag_cps:
        cp.wait()

    cnt = jnp.int32(0)
    for q in range(n_per):
        for l in range(0, CHUNK, L):
            jv = (row0 + q) * CHUNK + l + iota_v
            wv = wb.at[q, pl.ds(l, L)][...]
            local = wv - half0
            lose = (wv != jv) & (local >= 0) & (local < H)
            mi = lose.astype(jnp.int32)
            pc = plsc.cumsum(mi)
            pos = cnt + pc - 1
            plsc.store_scatter(jl, [pos >> 7, pos & 127], jv, mask=lose)
            plsc.store_scatter(gl, [pos >> 7, pos & 127], wv, mask=lose)
            plsc.store_scatter(dl, [pos >> 7, pos & 127], wv, mask=lose)
            plsc.store_scatter(tl, [pos >> 7, pos & 127], local, mask=lose)
            l1 = lose & (pos < 32)
            zr = pos * 0
            plsc.store_scatter(jl1, [zr, pos], jv, mask=l1)
            plsc.store_scatter(gl1, [zr, pos], wv, mask=l1)
            plsc.store_scatter(dl1, [zr, pos], wv, mask=l1)
            plsc.store_scatter(tl1, [zr, pos], local, mask=l1)
            cnt = cnt + jnp.sum(mi)
    nblk = (cnt + CHUNK - 1) // CHUNK
    for cp in copy_cps:
        cp.wait()
    plsc.subcore_barrier()

    small = (cnt > 0) & (cnt <= 32)
    big = cnt > 32

    @pl.when(small)
    def _():
        pltpu.sync_copy(val_hbm.at[gl1.at[0]], vb0.at[pl.ds(0, 32)])
        pltpu.sync_copy(vb0.at[pl.ds(0, 32)], acc_sh.at[tl1.at[0]])

    for b in range(n_per):
        @pl.when(big & (b < nblk))
        def _():
            pltpu.sync_copy(val_hbm.at[gl.at[b]], vb0)
            pltpu.sync_copy(vb0, acc_sh.at[tl.at[b]])

    plsc.subcore_barrier()

    @pl.when(small)
    def _():
        pltpu.sync_copy(val_hbm.at[jl1.at[0]], vb1.at[pl.ds(0, 32)])
        pltpu.sync_copy(vb1.at[pl.ds(0, 32)], acc_sh.at[tl1.at[0]], add=True)

    for b in range(n_per):
        @pl.when(big & (b < nblk))
        def _():
            pltpu.sync_copy(val_hbm.at[jl.at[b]], vb1)
            pltpu.sync_copy(vb1, acc_sh.at[tl.at[b]], add=True)

    plsc.subcore_barrier()

    @pl.when(small)
    def _():
        pltpu.sync_copy(acc_sh.at[tl1.at[0]], vb2.at[pl.ds(0, 32)])
        pltpu.sync_copy(vb2.at[pl.ds(0, 32)], acc_hbm.at[dl1.at[0]])

    for b in range(n_per):
        @pl.when(big & (b < nblk))
        def _():
            pltpu.sync_copy(acc_sh.at[tl.at[b]], vb2)
            pltpu.sync_copy(vb2, acc_hbm.at[dl.at[b]])


@functools.partial(
    pl.kernel,
    out_type=jax.ShapeDtypeStruct((B, D), jnp.float32),
    mesh=_mesh,
    scratch_types=[
        pltpu.VMEM((ROWS // NW, CHUNK), jnp.int32),
        pltpu.VMEM((ROWS // NW, CHUNK), jnp.int32),
        pltpu.VMEM((CHUNK, D), jnp.float32),
        pltpu.VMEM((CHUNK, D), jnp.float32),
        pltpu.VMEM((CHUNK, D), jnp.float32),
        pltpu.VMEM((CHUNK, D), jnp.float32),
        pltpu.SemaphoreType.DMA,
        pltpu.SemaphoreType.DMA,
        pltpu.SemaphoreType.DMA,
    ],
)
def _combine_kernel(idx_hbm, tag_hbm, mem_hbm, acc_hbm, out_hbm,
                    idxb, wb, mb0, mb1, ab0, ab1, sem_in, sem_g, sem_out):
    wid = _wid()
    n_per = ROWS // NW
    row0 = wid * n_per

    pltpu.sync_copy(idx_hbm.at[pl.ds(row0, n_per)], idxb)
    tag_cps = [
        pltpu.async_copy(tag_hbm.at[idxb.at[q]], wb.at[q], sem_in)
        for q in range(n_per)
    ]
    for cp in tag_cps:
        cp.wait()

    mbufs = (mb0, mb1)
    abufs = (ab0, ab1)

    def fire(q):
        b = q % 2
        return (pltpu.async_copy(mem_hbm.at[idxb.at[q]], mbufs[b], sem_g),
                pltpu.async_copy(acc_hbm.at[wb.at[q]], abufs[b], sem_g))

    g_cps = [None] * n_per
    out_cps = [None] * n_per
    g_cps[0] = fire(0)
    for q in range(n_per):
        mb, ab = mbufs[q % 2], abufs[q % 2]
        if q + 1 < n_per:
            if q >= 1:
                out_cps[q - 1].wait()
            g_cps[q + 1] = fire(q + 1)
        for cp in g_cps[q]:
            cp.wait()

        @pl.loop(0, CHUNK)
        def _(r):
            for l in range(0, D, L):
                sl = (r, pl.ds(l, L))
                mb.at[sl][...] = mb.at[sl][...] + ab.at[sl][...] * DECAY_F

        out_cps[q] = pltpu.async_copy(
            mb, out_hbm.at[pl.ds((row0 + q) * CHUNK, CHUNK)], sem_out)
    out_cps[n_per - 2].wait()
    out_cps[n_per - 1].wait()


def kernel(mem, idx, val):
    idx2 = jnp.reshape(idx.astype(jnp.int32), (ROWS, CHUNK))
    tags = _tag_kernel(idx2)
    acc = _acc_kernel(idx2, tags, val)
    return _combine_kernel(idx2, tags, mem, acc)


# speedup vs baseline: 6.8706x; 1.0261x over previous
import dataclasses
import functools

import jax
import jax.numpy as jnp
from jax import lax
from jax.experimental import pallas as pl
from jax.experimental.pallas import tpu as pltpu
from jax.experimental.pallas import tpu_sc as plsc

M = 1000000
D = 128
B = 16384
DECAY_F = 0.95

NC = 2
NS = 16
L = 16
NW = NC * NS
CHUNK = 128
ROWS = B // CHUNK
H = B // NC
TRASH = H

_mesh = plsc.VectorSubcoreMesh(core_axis_name="c", subcore_axis_name="s")

_no_layout_cp = pltpu.CompilerParams()
if "needs_layout_passes" in pltpu.CompilerParams.__dataclass_fields__:
    _no_layout_cp = dataclasses.replace(_no_layout_cp, needs_layout_passes=False)


def _wid():
    return lax.axis_index("s") * NC + lax.axis_index("c")


@functools.partial(
    pl.kernel,
    out_type=jax.ShapeDtypeStruct((M,), jnp.int32),
    mesh=_mesh,
    scratch_types=[
        pltpu.VMEM((ROWS // NW, CHUNK), jnp.int32),
        pltpu.VMEM((ROWS // NW, CHUNK), jnp.int32),
        pltpu.SemaphoreType.DMA,
        pltpu.SemaphoreType.DMA,
    ],
)
def _tag_kernel(idx_hbm, tag_hbm, idxb, jb, sem_in, sem_sc):
    wid = _wid()
    n_per = ROWS // NW
    row0 = wid * n_per

    cp = pltpu.async_copy(idx_hbm.at[pl.ds(row0, n_per)], idxb, sem_in)
    for q in range(n_per):
        j0 = (row0 + q) * CHUNK
        for l in range(0, CHUNK, L):
            jb.at[q, pl.ds(l, L)][...] = j0 + l + lax.iota(jnp.int32, L)
    cp.wait()
    cps = [
        pltpu.async_copy(jb.at[q], tag_hbm.at[idxb.at[q]], sem_sc)
        for q in range(n_per)
    ]
    for cp in cps:
        cp.wait()


@functools.partial(
    pl.kernel,
    out_type=jax.ShapeDtypeStruct((B + CHUNK, D), jnp.float32),
    mesh=_mesh,
    scratch_types=[
        pltpu.VMEM((CHUNK, D), jnp.float32),
        pltpu.VMEM((CHUNK, D), jnp.float32),
        pltpu.VMEM((CHUNK, D), jnp.float32),
        pltpu.VMEM((ROWS // NS, CHUNK), jnp.int32),
        pltpu.VMEM((ROWS // NS, CHUNK), jnp.int32),
        pltpu.VMEM((ROWS // NS, CHUNK), jnp.int32),
        pltpu.VMEM((ROWS // NS, CHUNK), jnp.int32),
        pltpu.VMEM((ROWS // NS, CHUNK), jnp.int32),
        pltpu.VMEM((ROWS // NS, CHUNK), jnp.int32),
        pltpu.VMEM((1, 32), jnp.int32),
        pltpu.VMEM((1, 32), jnp.int32),
        pltpu.VMEM((1, 32), jnp.int32),
        pltpu.VMEM((1, 32), jnp.int32),
        pltpu.VMEM_SHARED((H + CHUNK, D), jnp.float32),
        pltpu.SemaphoreType.DMA,
        pltpu.SemaphoreType.DMA,
        pltpu.SemaphoreType.DMA,
    ],
    compiler_params=_no_layout_cp,
)
def _acc_kernel(idx_hbm, tag_hbm, val_hbm, acc_hbm,
                vb0, vb1, vb2, idxb, wb, jl, gl, tl, dl,
                jl1, gl1, dl1, tl1, acc_sh,
                sem_in, sem_cp, sem_wr):
    c = lax.axis_index("c")
    s = lax.axis_index("s")
    half0 = c * H
    n_per = ROWS // NS
    row0 = s * n_per
    crows = H // NS

    cp_idx = pltpu.async_copy(idx_hbm.at[pl.ds(row0, n_per)], idxb, sem_in)

    vbufs = (vb0, vb1, vb2)
    n_cp = crows // CHUNK
    cin = [None] * n_cp
    cout = [None] * n_cp
    for q in range(min(3, n_cp)):
        cin[q] = pltpu.async_copy(
            val_hbm.at[pl.ds(half0 + s * crows + q * CHUNK, CHUNK)],
            vbufs[q % 3], sem_cp)
    for q in range(n_cp):
        if q >= 3:
            cout[q - 3].wait()
            cin[q] = pltpu.async_copy(
                val_hbm.at[pl.ds(half0 + s * crows + q * CHUNK, CHUNK)],
                vbufs[q % 3], sem_cp)
        cin[q].wait()
        cout[q] = pltpu.async_copy(
            vbufs[q % 3],
            acc_hbm.at[pl.ds(half0 + s * crows + q * CHUNK, CHUNK)], sem_wr)
    copy_cps = [cout[q] for q in range(max(0, n_cp - 3), n_cp)]
    cp_idx.wait()
    tag_cps = [
        pltpu.async_copy(tag_hbm.at[idxb.at[q]], wb.at[q], sem_in)
        for q in range(n_per)
    ]

    iota_v = lax.iota(jnp.int32, L)
    iota_v = lax.iota(jnp.int32, L)
    for q in range(n_per):
        for l in range(0, CHUNK, L):
            pad = l + iota_v
            jl.at[q, pl.ds(l, L)][...] = pad
            gl.at[q, pl.ds(l, L)][...] = pad
            dl.at[q, pl.ds(l, L)][...] = pad + B
            tl.at[q, pl.ds(l, L)][...] = pad + TRASH
    for l in range(0, 32, L):
        pad = l + iota_v
        jl1.at[0, pl.ds(l, L)][...] = pad
        gl1.at[0, pl.ds(l, L)][...] = pad
        dl1.at[0, pl.ds(l, L)][...] = pad + B
        tl1.at[0, pl.ds(l, L)][...] = pad + TRASH
    for cp in tag_cps:
        cp.wait()

    cnt = jnp.int32(0)
    for q in range(n_per):
        for l in range(0, CHUNK, L):
            jv = (row0 + q) * CHUNK + l + iota_v
            wv = wb.at[q, pl.ds(l, L)][...]
            local = wv - half0
            lose = (wv != jv) & (local >= 0) & (local < H)
            mi = lose.astype(jnp.int32)
            pc = plsc.cumsum(mi)
            pos = cnt + pc - 1
            plsc.store_scatter(jl, [pos >> 7, pos & 127], jv, mask=lose)
            plsc.store_scatter(gl, [pos >> 7, pos & 127], wv, mask=lose)
            plsc.store_scatter(dl, [pos >> 7, pos & 127], wv, mask=lose)
            plsc.store_scatter(tl, [pos >> 7, pos & 127], local, mask=lose)
            l1 = lose & (pos < 32)
            zr = pos * 0
            plsc.store_scatter(jl1, [zr, pos], jv, mask=l1)
            plsc.store_scatter(gl1, [zr, pos], wv, mask=l1)
            plsc.store_scatter(dl1, [zr, pos], wv, mask=l1)
            plsc.store_scatter(tl1, [zr, pos], local, mask=l1)
            cnt = cnt + jnp.sum(mi)
    nblk = (cnt + CHUNK - 1) // CHUNK
    for cp in copy_cps:
        cp.wait()
    small = (cnt > 0) & (cnt <= 32)
    big = cnt > 32
    g1i = pltpu.async_copy(val_hbm.at[gl1.at[0]], vb0.at[pl.ds(0, 32)],
                           sem_in)
    g1a = pltpu.async_copy(val_hbm.at[jl1.at[0]], vb1.at[pl.ds(0, 32)],
                           sem_in)
    plsc.subcore_barrier()

    g1i.wait()

    @pl.when(small)
    def _():
        pltpu.sync_copy(vb0.at[pl.ds(0, 32)], acc_sh.at[tl1.at[0]])

    for b in range(n_per):
        @pl.when(big & (b < nblk))
        def _():
            pltpu.sync_copy(val_hbm.at[gl.at[b]], vb0)
            pltpu.sync_copy(vb0, acc_sh.at[tl.at[b]])

    plsc.subcore_barrier()

    g1a.wait()

    @pl.when(small)
    def _():
        pltpu.sync_copy(vb1.at[pl.ds(0, 32)], acc_sh.at[tl1.at[0]], add=True)

    for b in range(n_per):
        @pl.when(big & (b < nblk))
        def _():
            pltpu.sync_copy(val_hbm.at[jl.at[b]], vb1)
            pltpu.sync_copy(vb1, acc_sh.at[tl.at[b]], add=True)

    plsc.subcore_barrier()

    @pl.when(small)
    def _():
        pltpu.sync_copy(acc_sh.at[tl1.at[0]], vb2.at[pl.ds(0, 32)])
        pltpu.sync_copy(vb2.at[pl.ds(0, 32)], acc_hbm.at[dl1.at[0]])

    for b in range(n_per):
        @pl.when(big & (b < nblk))
        def _():
            pltpu.sync_copy(acc_sh.at[tl.at[b]], vb2)
            pltpu.sync_copy(vb2, acc_hbm.at[dl.at[b]])


@functools.partial(
    pl.kernel,
    out_type=jax.ShapeDtypeStruct((B, D), jnp.float32),
    mesh=_mesh,
    scratch_types=[
        pltpu.VMEM((ROWS // NW, CHUNK), jnp.int32),
        pltpu.VMEM((ROWS // NW, CHUNK), jnp.int32),
        pltpu.VMEM((CHUNK, D), jnp.float32),
        pltpu.VMEM((CHUNK, D), jnp.float32),
        pltpu.VMEM((CHUNK, D), jnp.float32),
        pltpu.VMEM((CHUNK, D), jnp.float32),
        pltpu.SemaphoreType.DMA,
        pltpu.SemaphoreType.DMA,
        pltpu.SemaphoreType.DMA,
    ],
)
def _combine_kernel(idx_hbm, tag_hbm, mem_hbm, acc_hbm, out_hbm,
                    idxb, wb, mb0, mb1, ab0, ab1, sem_in, sem_g, sem_out):
    wid = _wid()
    n_per = ROWS // NW
    row0 = wid * n_per

    pltpu.sync_copy(idx_hbm.at[pl.ds(row0, n_per)], idxb)
    tag_cps = [
        pltpu.async_copy(tag_hbm.at[idxb.at[q]], wb.at[q], sem_in)
        for q in range(n_per)
    ]
    for cp in tag_cps:
        cp.wait()

    mbufs = (mb0, mb1)
    abufs = (ab0, ab1)

    def fire(q):
        b = q % 2
        return (pltpu.async_copy(mem_hbm.at[idxb.at[q]], mbufs[b], sem_g),
                pltpu.async_copy(acc_hbm.at[wb.at[q]], abufs[b], sem_g))

    g_cps = [None] * n_per
    out_cps = [None] * n_per
    g_cps[0] = fire(0)
    for q in range(n_per):
        mb, ab = mbufs[q % 2], abufs[q % 2]
        if q + 1 < n_per:
            if q >= 1:
                out_cps[q - 1].wait()
            g_cps[q + 1] = fire(q + 1)
        for cp in g_cps[q]:
            cp.wait()

        @pl.loop(0, CHUNK)
        def _(r):
            for l in range(0, D, L):
                sl = (r, pl.ds(l, L))
                mb.at[sl][...] = mb.at[sl][...] + ab.at[sl][...] * DECAY_F

        out_cps[q] = pltpu.async_copy(
            mb, out_hbm.at[pl.ds((row0 + q) * CHUNK, CHUNK)], sem_out)
    out_cps[n_per - 2].wait()
    out_cps[n_per - 1].wait()


def kernel(mem, idx, val):
    idx2 = jnp.reshape(idx.astype(jnp.int32), (ROWS, CHUNK))
    tags = _tag_kernel(idx2)
    acc = _acc_kernel(idx2, tags, val)
    return _combine_kernel(idx2, tags, mem, acc)


# speedup vs baseline: 6.9373x; 1.0097x over previous
import dataclasses
import functools

import jax
import jax.numpy as jnp
from jax import lax
from jax.experimental import pallas as pl
from jax.experimental.pallas import tpu as pltpu
from jax.experimental.pallas import tpu_sc as plsc

M = 1000000
D = 128
B = 16384
DECAY_F = 0.95

NC = 2
NS = 16
L = 16
NW = NC * NS
CHUNK = 128
ROWS = B // CHUNK
H = B // NC
TRASH = H

_mesh = plsc.VectorSubcoreMesh(core_axis_name="c", subcore_axis_name="s")

_no_layout_cp = pltpu.CompilerParams()
if "needs_layout_passes" in pltpu.CompilerParams.__dataclass_fields__:
    _no_layout_cp = dataclasses.replace(_no_layout_cp, needs_layout_passes=False)


def _wid():
    return lax.axis_index("s") * NC + lax.axis_index("c")


@functools.partial(
    pl.kernel,
    out_type=jax.ShapeDtypeStruct((M,), jnp.int32),
    mesh=_mesh,
    scratch_types=[
        pltpu.VMEM((ROWS // NW, CHUNK), jnp.int32),
        pltpu.VMEM((ROWS // NW, CHUNK), jnp.int32),
        pltpu.SemaphoreType.DMA,
        pltpu.SemaphoreType.DMA,
    ],
)
def _tag_kernel(idx_hbm, tag_hbm, idxb, jb, sem_in, sem_sc):
    wid = _wid()
    n_per = ROWS // NW
    row0 = wid * n_per

    cp = pltpu.async_copy(idx_hbm.at[pl.ds(row0, n_per)], idxb, sem_in)
    for q in range(n_per):
        j0 = (row0 + q) * CHUNK
        for l in range(0, CHUNK, L):
            jb.at[q, pl.ds(l, L)][...] = j0 + l + lax.iota(jnp.int32, L)
    cp.wait()
    cps = [
        pltpu.async_copy(jb.at[q], tag_hbm.at[idxb.at[q]], sem_sc)
        for q in range(n_per)
    ]
    for cp in cps:
        cp.wait()


@functools.partial(
    pl.kernel,
    out_type=jax.ShapeDtypeStruct((B + CHUNK, D), jnp.float32),
    mesh=_mesh,
    scratch_types=[
        pltpu.VMEM((CHUNK, D), jnp.float32),
        pltpu.VMEM((CHUNK, D), jnp.float32),
        pltpu.VMEM((CHUNK, D), jnp.float32),
        pltpu.VMEM((ROWS // NS, CHUNK), jnp.int32),
        pltpu.VMEM((ROWS // NS, CHUNK), jnp.int32),
        pltpu.VMEM((ROWS // NS, CHUNK), jnp.int32),
        pltpu.VMEM((ROWS // NS, CHUNK), jnp.int32),
        pltpu.VMEM((ROWS // NS, CHUNK), jnp.int32),
        pltpu.VMEM((ROWS // NS, CHUNK), jnp.int32),
        pltpu.VMEM((1, 32), jnp.int32),
        pltpu.VMEM((1, 32), jnp.int32),
        pltpu.VMEM((1, 32), jnp.int32),
        pltpu.VMEM((1, 32), jnp.int32),
        pltpu.VMEM_SHARED((H + CHUNK, D), jnp.float32),
        pltpu.SemaphoreType.DMA,
        pltpu.SemaphoreType.DMA,
        pltpu.SemaphoreType.DMA,
    ],
    compiler_params=_no_layout_cp,
)
def _acc_kernel(idx_hbm, tag_hbm, val_hbm, acc_hbm,
                vb0, vb1, vb2, idxb, wb, jl, gl, tl, dl,
                jl1, gl1, dl1, tl1, acc_sh,
                sem_in, sem_cp, sem_wr):
    c = lax.axis_index("c")
    s = lax.axis_index("s")
    half0 = c * H
    n_per = ROWS // NS
    row0 = s * n_per
    crows = H // NS

    cp_idx = pltpu.async_copy(idx_hbm.at[pl.ds(row0, n_per)], idxb, sem_in)

    vbufs = (vb0, vb1, vb2)
    n_cp = crows // CHUNK
    cin = [None] * n_cp
    cout = [None] * n_cp
    for q in range(min(3, n_cp)):
        cin[q] = pltpu.async_copy(
            val_hbm.at[pl.ds(half0 + s * crows + q * CHUNK, CHUNK)],
            vbufs[q % 3], sem_cp)
    cp_idx.wait()
    tag_cps = [
        pltpu.async_copy(tag_hbm.at[idxb.at[q]], wb.at[q], sem_in)
        for q in range(n_per)
    ]
    for q in range(n_cp):
        if q >= 3:
            cout[q - 3].wait()
            cin[q] = pltpu.async_copy(
                val_hbm.at[pl.ds(half0 + s * crows + q * CHUNK, CHUNK)],
                vbufs[q % 3], sem_cp)
        cin[q].wait()
        cout[q] = pltpu.async_copy(
            vbufs[q % 3],
            acc_hbm.at[pl.ds(half0 + s * crows + q * CHUNK, CHUNK)], sem_wr)
    copy_cps = [cout[q] for q in range(max(0, n_cp - 3), n_cp)]

    iota_v = lax.iota(jnp.int32, L)
    iota_v = lax.iota(jnp.int32, L)
    for q in range(n_per):
        for l in range(0, CHUNK, L):
            pad = l + iota_v
            jl.at[q, pl.ds(l, L)][...] = pad
            gl.at[q, pl.ds(l, L)][...] = pad
            dl.at[q, pl.ds(l, L)][...] = pad + B
            tl.at[q, pl.ds(l, L)][...] = pad + TRASH
    for l in range(0, 32, L):
        pad = l + iota_v
        jl1.at[0, pl.ds(l, L)][...] = pad
        gl1.at[0, pl.ds(l, L)][...] = pad
        dl1.at[0, pl.ds(l, L)][...] = pad + B
        tl1.at[0, pl.ds(l, L)][...] = pad + TRASH
    for cp in tag_cps:
        cp.wait()

    cnt = jnp.int32(0)
    for q in range(n_per):
        for l in range(0, CHUNK, L):
            jv = (row0 + q) * CHUNK + l + iota_v
            wv = wb.at[q, pl.ds(l, L)][...]
            local = wv - half0
            lose = (wv != jv) & (local >= 0) & (local < H)
            mi = lose.astype(jnp.int32)
            pc = plsc.cumsum(mi)
            pos = cnt + pc - 1
            plsc.store_scatter(jl, [pos >> 7, pos & 127], jv, mask=lose)
            plsc.store_scatter(gl, [pos >> 7, pos & 127], wv, mask=lose)
            plsc.store_scatter(dl, [pos >> 7, pos & 127], wv, mask=lose)
            plsc.store_scatter(tl, [pos >> 7, pos & 127], local, mask=lose)
            l1 = lose & (pos < 32)
            zr = pos * 0
            plsc.store_scatter(jl1, [zr, pos], jv, mask=l1)
            plsc.store_scatter(gl1, [zr, pos], wv, mask=l1)
            plsc.store_scatter(dl1, [zr, pos], wv, mask=l1)
            plsc.store_scatter(tl1, [zr, pos], local, mask=l1)
            cnt = cnt + jnp.sum(mi)
    nblk = (cnt + CHUNK - 1) // CHUNK
    for cp in copy_cps:
        cp.wait()
    small = (cnt > 0) & (cnt <= 32)
    big = cnt > 32
    g1i = pltpu.async_copy(val_hbm.at[gl1.at[0]], vb0.at[pl.ds(0, 32)],
                           sem_in)
    g1a = pltpu.async_copy(val_hbm.at[jl1.at[0]], vb1.at[pl.ds(0, 32)],
                           sem_in)
    plsc.subcore_barrier()

    g1i.wait()

    @pl.when(small)
    def _():
        pltpu.sync_copy(vb0.at[pl.ds(0, 32)], acc_sh.at[tl1.at[0]])

    for b in range(n_per):
        @pl.when(big & (b < nblk))
        def _():
            pltpu.sync_copy(val_hbm.at[gl.at[b]], vb0)
            pltpu.sync_copy(vb0, acc_sh.at[tl.at[b]])

    plsc.subcore_barrier()

    g1a.wait()

    @pl.when(small)
    def _():
        pltpu.sync_copy(vb1.at[pl.ds(0, 32)], acc_sh.at[tl1.at[0]], add=True)

    for b in range(n_per):
        @pl.when(big & (b < nblk))
        def _():
            pltpu.sync_copy(val_hbm.at[jl.at[b]], vb1)
            pltpu.sync_copy(vb1, acc_sh.at[tl.at[b]], add=True)

    plsc.subcore_barrier()

    @pl.when(small)
    def _():
        pltpu.sync_copy(acc_sh.at[tl1.at[0]], vb2.at[pl.ds(0, 32)])
        pltpu.sync_copy(vb2.at[pl.ds(0, 32)], acc_hbm.at[dl1.at[0]])

    for b in range(n_per):
        @pl.when(big & (b < nblk))
        def _():
            pltpu.sync_copy(acc_sh.at[tl.at[b]], vb2)
            pltpu.sync_copy(vb2, acc_hbm.at[dl.at[b]])


@functools.partial(
    pl.kernel,
    out_type=jax.ShapeDtypeStruct((B, D), jnp.float32),
    mesh=_mesh,
    scratch_types=[
        pltpu.VMEM((ROWS // NW, CHUNK), jnp.int32),
        pltpu.VMEM((ROWS // NW, CHUNK), jnp.int32),
        pltpu.VMEM((CHUNK, D), jnp.float32),
        pltpu.VMEM((CHUNK, D), jnp.float32),
        pltpu.VMEM((CHUNK, D), jnp.float32),
        pltpu.VMEM((CHUNK, D), jnp.float32),
        pltpu.SemaphoreType.DMA,
        pltpu.SemaphoreType.DMA,
        pltpu.SemaphoreType.DMA,
    ],
)
def _combine_kernel(idx_hbm, tag_hbm, mem_hbm, acc_hbm, out_hbm,
                    idxb, wb, mb0, mb1, ab0, ab1, sem_in, sem_g, sem_out):
    wid = _wid()
    n_per = ROWS // NW
    row0 = wid * n_per

    pltpu.sync_copy(idx_hbm.at[pl.ds(row0, n_per)], idxb)
    tag_cps = [
        pltpu.async_copy(tag_hbm.at[idxb.at[q]], wb.at[q], sem_in)
        for q in range(n_per)
    ]
    for cp in tag_cps:
        cp.wait()

    mbufs = (mb0, mb1)
    abufs = (ab0, ab1)

    def fire(q):
        b = q % 2
        return (pltpu.async_copy(mem_hbm.at[idxb.at[q]], mbufs[b], sem_g),
                pltpu.async_copy(acc_hbm.at[wb.at[q]], abufs[b], sem_g))

    g_cps = [None] * n_per
    out_cps = [None] * n_per
    g_cps[0] = fire(0)
    for q in range(n_per):
        mb, ab = mbufs[q % 2], abufs[q % 2]
        if q + 1 < n_per:
            if q >= 1:
                out_cps[q - 1].wait()
            g_cps[q + 1] = fire(q + 1)
        for cp in g_cps[q]:
            cp.wait()

        @pl.loop(0, CHUNK)
        def _(r):
            for l in range(0, D, L):
                sl = (r, pl.ds(l, L))
                mb.at[sl][...] = mb.at[sl][...] + ab.at[sl][...] * DECAY_F

        out_cps[q] = pltpu.async_copy(
            mb, out_hbm.at[pl.ds((row0 + q) * CHUNK, CHUNK)], sem_out)
    out_cps[n_per - 2].wait()
    out_cps[n_per - 1].wait()


def kernel(mem, idx, val):
    idx2 = jnp.reshape(idx.astype(jnp.int32), (ROWS, CHUNK))
    tags = _tag_kernel(idx2)
    acc = _acc_kernel(idx2, tags, val)
    return _combine_kernel(idx2, tags, mem, acc)
